# baseline (speedup 1.0000x reference)
import functools

import jax
import jax.numpy as jnp
from jax import lax
from jax.experimental import pallas as pl
from jax.experimental.pallas import tpu as pltpu
from jax.experimental.pallas import tpu_sc as plsc

NC = 2
NS = 16
CH = 128
EC = 64
SB = 16

f32 = jnp.float32
i32 = jnp.int32


def _ceil_to(v, m):
    return (v + m - 1) // m * m


def _sc_mesh():
    return plsc.VectorSubcoreMesh(core_axis_name="c", subcore_axis_name="s")


def _mm_body(a_ref, w_ref, o_ref):
    o_ref[...] = jnp.dot(a_ref[...], w_ref[...], preferred_element_type=f32)


def _matmul(a, w, bm=256):
    m, k = a.shape
    _, n = w.shape
    return pl.pallas_call(
        _mm_body,
        grid=(m // bm,),
        in_specs=[pl.BlockSpec((bm, k), lambda i: (i, 0)),
                  pl.BlockSpec((k, n), lambda i: (0, 0))],
        out_specs=pl.BlockSpec((bm, n), lambda i: (i, 0)),
        out_shape=jax.ShapeDtypeStruct((m, n), f32),
    )(a, w)


def _bf16_bits(x):
    b = lax.bitcast_convert_type(x, jnp.uint32)
    return (b + 0x7FFF + ((b >> 16) & 1)) >> 16


def _pack_pair(lo_f32, hi_f32):
    lo = _bf16_bits(lo_f32)
    hi = _bf16_bits(hi_f32)
    return lax.bitcast_convert_type(lo | (hi << 16), i32)


def _tcb_body(deg_ref, h1_ref, g0_ref, g1_ref, w4_ref, tp_ref, zp_ref,
              di_ref):
    d = h1_ref.shape[1]
    dh = d // NC
    hq = dh // 2
    deg = deg_ref[0] + deg_ref[1] + 1.0
    dinv = lax.rsqrt(deg)[:, None]
    di_ref[...] = dinv
    a = h1_ref[...] * dinv
    z = g0_ref[...] + g1_ref[...]
    zp_ref[...] = z
    b = jnp.dot(z, w4_ref[...], preferred_element_type=f32)
    rows = []
    for cc in range(NC):
        pa = _pack_pair(a[:, cc * dh:cc * dh + hq],
                        a[:, cc * dh + hq:(cc + 1) * dh])
        pb = _pack_pair(b[:, cc * dh:cc * dh + hq],
                        b[:, cc * dh + hq:(cc + 1) * dh])
        rows.append(jnp.concatenate([pa, pb], axis=1))
    tp_ref[...] = jnp.stack(rows, axis=1)


def _tc_b(deg2, h1, g0, g1, w4, npad):
    bm = 256
    d = h1.shape[1]
    dh = d // NC
    return pl.pallas_call(
        _tcb_body,
        grid=(npad // bm,),
        in_specs=[
            pl.BlockSpec((NC, bm), lambda i: (0, i)),
            pl.BlockSpec((bm, d), lambda i: (i, 0)),
            pl.BlockSpec((bm, d), lambda i: (i, 0)),
            pl.BlockSpec((bm, d), lambda i: (i, 0)),
            pl.BlockSpec((d, d), lambda i: (0, 0)),
        ],
        out_specs=[
            pl.BlockSpec((bm, NC, dh), lambda i: (i, 0, 0)),
            pl.BlockSpec((bm, d), lambda i: (i, 0)),
            pl.BlockSpec((bm, 1), lambda i: (i, 0)),
        ],
        out_shape=[
            jax.ShapeDtypeStruct((npad, NC, dh), i32),
            jax.ShapeDtypeStruct((npad, d), f32),
            jax.ShapeDtypeStruct((npad, 1), f32),
        ],
    )(deg2, h1, g0, g1, w4)


def _tcc_body(a0_ref, a1_ref, c0_ref, c1_ref, di_ref, h1_ref, zp_ref,
              b1_ref, gam_ref, bet_ref, wsq_ref, o_ref):
    d = h1_ref.shape[1]
    dinv = di_ref[...]
    xa = jnp.concatenate([a0_ref[0], a1_ref[0]], axis=-1)
    xp = dinv * xa + (dinv * dinv) * h1_ref[...] + b1_ref[...]
    zp = zp_ref[...]
    za = jnp.concatenate([c0_ref[0], c1_ref[0]], axis=-1)
    w = wsq_ref[...]
    gam = gam_ref[...]
    bet = bet_ref[...]

    def ln(t):
        mu = jnp.mean(t, axis=-1, keepdims=True)
        tc = t - mu
        var = jnp.mean(tc * tc, axis=-1, keepdims=True)
        return tc * lax.rsqrt(var + 1e-5) * gam + bet

    acc = jnp.dot(ln(xp), w[0:d], preferred_element_type=f32)
    acc = acc + jnp.dot(ln(zp), w[d:2 * d], preferred_element_type=f32)
    acc = acc + jnp.dot(ln(za), w[2 * d:3 * d], preferred_element_type=f32)
    o_ref[...] = acc


def _tc_c(acc4, dinv1, h1, zp, b1r, gamr, betr, wsq, n):
    br = 200
    d = h1.shape[1]
    dh = d // NC

    def qspec(q):
        return pl.BlockSpec((1, br, dh), lambda i, q=q: (q, i, 0))

    return pl.pallas_call(
        _tcc_body,
        grid=(n // br,),
        in_specs=[
            qspec(0), qspec(1), qspec(2), qspec(3),
            pl.BlockSpec((br, 1), lambda i: (i, 0)),
            pl.BlockSpec((br, d), lambda i: (i, 0)),
            pl.BlockSpec((br, d), lambda i: (i, 0)),
            pl.BlockSpec((1, d), lambda i: (0, 0)),
            pl.BlockSpec((1, d), lambda i: (0, 0)),
            pl.BlockSpec((1, d), lambda i: (0, 0)),
            pl.BlockSpec((3 * d, d), lambda i: (0, 0)),
        ],
        out_specs=pl.BlockSpec((br, d), lambda i: (i, 0)),
        out_shape=jax.ShapeDtypeStruct((n, d), f32),
    )(acc4, acc4, acc4, acc4, dinv1, h1, zp, b1r, gamr, betr, wsq)


def _sc_deg(dst2d, npad, epad):
    per = npad // NS
    nrows = epad // CH
    nch = nrows // (NC * NS)
    R = 4

    @functools.partial(
        pl.kernel,
        out_type=jax.ShapeDtypeStruct((NC * npad,), f32),
        mesh=_sc_mesh(),
        scratch_types=[
            pltpu.VMEM((nch, CH), i32),
            pltpu.VMEM((CH,), f32),
            pltpu.VMEM_SHARED((npad,), f32),
        ] + [pltpu.SemaphoreType.DMA] * R,
    )
    def body(dst_hbm, out_hbm, didx, ones_v, acc_sh, s0, s1, s2, s3):
        c = lax.axis_index("c")
        s = lax.axis_index("s")
        ssems = (s0, s1, s2, s3)
        w = c * NS + s
        pltpu.sync_copy(dst_hbm.at[pl.ds(w * nch, nch), :], didx)
        for q in range(CH // 16):
            ones_v[pl.ds(q * 16, 16)] = jnp.zeros((16,), f32)
        for kk in range(per // CH):
            pltpu.sync_copy(ones_v, acc_sh.at[pl.ds(s * per + kk * CH, CH)])
        for q in range(CH // 16):
            ones_v[pl.ds(q * 16, 16)] = jnp.ones((16,), f32)
        plsc.subcore_barrier()

        def fire(r, j):
            pltpu.async_copy(ones_v, acc_sh.at[didx.at[j]], ssems[r], add=True)

        def wait(r):
            pltpu.make_async_copy(ones_v, acc_sh.at[didx.at[0]],
                                  ssems[r]).wait()

        def step(j4, carry):
            for q in range(R):
                j = j4 * R + q

                @pl.when(j4 > 0)
                def _():
                    wait(q)

                fire(q, j)
            return carry

        lax.fori_loop(0, nch // R, step, 0)
        for r in range(R):
            wait(r)
        plsc.subcore_barrier()
        pltpu.sync_copy(acc_sh.at[pl.ds(s * per, per)],
                        out_hbm.at[pl.ds(c * npad + s * per, per)])

    return body(dst2d)


def _sc_bcgather(h2, a0p, a1p, npad):
    d = h2.shape[1]
    g = 64
    rpt = npad // (NC * NS)
    jobs_per_stream = rpt // g
    nj = 2 * jobs_per_stream
    R, G = 4, 2

    @functools.partial(
        pl.kernel,
        out_type=[jax.ShapeDtypeStruct((npad, d), f32),
                  jax.ShapeDtypeStruct((npad, d), f32)],
        mesh=_sc_mesh(),
        scratch_types=[
            pltpu.VMEM((rpt,), i32),
            pltpu.VMEM((rpt,), i32),
            pltpu.VMEM((R * g, d), f32),
        ] + [pltpu.SemaphoreType.DMA] * (2 * R),
    )
    def body(h2_hbm, a0_hbm, a1_hbm, g0_hbm, g1_hbm, i0, i1, ring,
             ga0, ga1, ga2, ga3, wa0, wa1, wa2, wa3):
        c = lax.axis_index("c")
        s = lax.axis_index("s")
        gsems = (ga0, ga1, ga2, ga3)
        wsems = (wa0, wa1, wa2, wa3)
        w = s * NC + c
        pltpu.sync_copy(a0_hbm.at[pl.ds(w * rpt, rpt)], i0)
        pltpu.sync_copy(a1_hbm.at[pl.ds(w * rpt, rpt)], i1)

        def slot(r):
            return ring.at[pl.ds(r * g, g), :]

        def job_refs(j):
            if j < jobs_per_stream:
                return i0.at[pl.ds(j * g, g)], g0_hbm, j
            return i1.at[pl.ds((j - jobs_per_stream) * g, g)], g1_hbm, \
                j - jobs_per_stream

        def fire_gather(r, j):
            idx, _, _ = job_refs(j)
            pltpu.async_copy(h2_hbm.at[idx], slot(r), gsems[r])

        def wait_gather(r, j):
            idx, _, _ = job_refs(j)
            pltpu.make_async_copy(h2_hbm.at[idx], slot(r), gsems[r]).wait()

        def out_rows(j):
            _, out, jj = job_refs(j)
            return out.at[pl.ds(w * rpt + jj * g, g), :]

        def fire_write(r, j):
            pltpu.async_copy(slot(r), out_rows(j), wsems[r])

        def wait_write(r, j):
            pltpu.make_async_copy(slot(r), out_rows(j), wsems[r]).wait()

        for j in range(G):
            fire_gather(j % R, j)
        for j in range(nj):
            r = j % R
            wait_gather(r, j)
            fire_write(r, j)
            jn = j + G
            if jn < nj:
                rn = jn % R
                if jn >= R:
                    wait_write(rn, jn - R)
                fire_gather(rn, jn)
        for j in range(nj - R, nj):
            wait_write(j % R, j)

    return body(h2, a0p, a1p)


def _sc_edge2(srcq2d, dst2d, tpflat, npad, epad):
    dh = 128
    hq = dh // 2
    per = npad // NS
    nrows = epad // EC
    nch = nrows // NS
    nsup = nch // SB
    R, G = 4, 3

    @functools.partial(
        pl.kernel,
        out_type=[jax.ShapeDtypeStruct((2 * NC * npad, dh), f32),
                  jax.ShapeDtypeStruct((NC * epad, dh), i32)],
        mesh=_sc_mesh(),
        scratch_types=[
            pltpu.VMEM((2 * SB, EC), i32),
            pltpu.VMEM((2 * SB, EC), i32),
            pltpu.VMEM((R * EC, dh), i32),
            pltpu.VMEM((EC, dh), f32),
            pltpu.VMEM_SHARED((npad, dh), f32),
        ] + [pltpu.SemaphoreType.DMA] * 13,
    )
    def body(srcq_hbm, dst_hbm, tp_hbm, out_hbm, sp_hbm,
             gidx, didx, ring, conv, acc_sh,
             g0, g1, g2, g3, p0, p1, p2, p3, sc0, ig0, ig1, id0, id1):
        c = lax.axis_index("c")
        s = lax.axis_index("s")
        gsems = (g0, g1, g2, g3)
        psems = (p0, p1, p2, p3)
        igsems = (ig0, ig1)
        idsems = (id0, id1)

        def slot(r):
            return ring.at[pl.ds(r * EC, EC), :]

        def spill_rows(j):
            return sp_hbm.at[pl.ds(c * epad + (s * nch + j) * EC, EC), :]

        def fire_spill(r, j):
            pltpu.async_copy(slot(r), spill_rows(j), psems[r])

        def wait_spill(r, j):
            pltpu.make_async_copy(slot(r), spill_rows(j), psems[r]).wait()

        def fire_scatter(row_sel):
            pltpu.async_copy(conv, acc_sh.at[didx.at[row_sel]], sc0, add=True)

        def wait_scatter(row_sel):
            pltpu.make_async_copy(conv, acc_sh.at[didx.at[row_sel]],
                                  sc0).wait()

        def unpack(r, lane0):
            def row(i, carry):
                for q in range(hq // 16):
                    v = ring[r * EC + i, pl.ds(lane0 + q * 16, 16)]
                    lo = lax.bitcast_convert_type(v << 16, f32)
                    hi = lax.bitcast_convert_type(v & jnp.int32(-65536), f32)
                    conv[i, pl.ds(q * 16, 16)] = lo
                    conv[i, pl.ds(hq + q * 16, 16)] = hi
                return carry

            lax.fori_loop(0, EC, row, 0)

        def zero_acc():
            def zrow(i, carry):
                for qq in range(dh // 16):
                    conv[i, pl.ds(qq * 16, 16)] = jnp.zeros((16,), f32)
                return carry

            lax.fori_loop(0, EC, zrow, 0)
            for kk in range(per // EC):
                pltpu.sync_copy(conv,
                                acc_sh.at[pl.ds(s * per + kk * EC, EC), :])
            plsc.subcore_barrier()

        def stage_idx(b, u, sync, phase1):
            ds_ = dst_hbm.at[pl.ds(s * nch + u * SB, SB), :]
            dv = didx.at[pl.ds(b * SB, SB), :]
            if sync:
                pltpu.sync_copy(ds_, dv)
            else:
                pltpu.async_copy(ds_, dv, idsems[b])
            if phase1:
                gs = srcq_hbm.at[pl.ds(c * nrows + s * nch + u * SB, SB), :]
                gv = gidx.at[pl.ds(b * SB, SB), :]
                if sync:
                    pltpu.sync_copy(gs, gv)
                else:
                    pltpu.async_copy(gs, gv, igsems[b])

        def wait_idx(b, phase1):
            ds_ = dst_hbm.at[pl.ds(s * nch, SB), :]
            dv = didx.at[pl.ds(b * SB, SB), :]
            pltpu.make_async_copy(ds_, dv, idsems[b]).wait()
            if phase1:
                gs = srcq_hbm.at[pl.ds(c * nrows, SB), :]
                gv = gidx.at[pl.ds(b * SB, SB), :]
                pltpu.make_async_copy(gs, gv, igsems[b]).wait()

        def fire_fetch(r, j, row_sel, phase1):
            if phase1:
                pltpu.async_copy(tp_hbm.at[gidx.at[row_sel]], slot(r),
                                 gsems[r])
            else:
                pltpu.async_copy(spill_rows(j), slot(r), gsems[r])

        def wait_fetch(r, j, row_sel, phase1):
            if phase1:
                pltpu.make_async_copy(tp_hbm.at[gidx.at[row_sel]], slot(r),
                                      gsems[r]).wait()
            else:
                pltpu.make_async_copy(spill_rows(j), slot(r), gsems[r]).wait()

        def run_phase(phase1):
            k = (0 if phase1 else NC) + c
            lane0 = 0 if phase1 else hq
            zero_acc()
            stage_idx(0, 0, True, phase1)
            for j in range(G):
                fire_fetch(j % R, j, j, phase1)

            def pair_step(u2, carry):
                for half in range(2):
                    u = u2 * 2 + half
                    b = half
                    bn = 1 - half
                    for jj in range(SB):
                        j = u * SB + jj
                        r = jj % R
                        row = b * SB + jj
                        if jj == 2:
                            @pl.when(u + 1 < nsup)
                            def _():
                                stage_idx(bn, u + 1, False, phase1)
                        wait_fetch(r, j, row, phase1)
                        fire_spill_maybe = phase1
                        if fire_spill_maybe:
                            fire_spill(r, j)

                        @pl.when(j > 0)
                        def _():
                            wait_scatter(row)

                        unpack(r, lane0)
                        fire_scatter(row)
                        jn = j + G
                        rn = (jj + G) % R
                        if jj < SB - G:
                            rown = b * SB + (jj + G)
                            crosses = False
                        else:
                            rown = bn * SB + (jj + G - SB)
                            crosses = True
                        if crosses:
                            @pl.when(jn < nch)
                            def _():
                                if phase1:
                                    wait_spill(rn, jn)
                                if jj == SB - G:
                                    wait_idx(bn, phase1)
                                fire_fetch(rn, jn, rown, phase1)
                        else:
                            @pl.when(jn >= R)
                            def _():
                                if phase1:
                                    wait_spill(rn, jn)

                            fire_fetch(rn, jn, rown, phase1)
                return carry

            lax.fori_loop(0, nsup // 2, pair_step, 0)
            wait_scatter(0)
            if phase1:
                for r in range(R):
                    wait_spill(r, 0)
            plsc.subcore_barrier()
            pltpu.sync_copy(acc_sh.at[pl.ds(s * per, per), :],
                            out_hbm.at[pl.ds(k * npad + s * per, per), :])

        run_phase(True)
        run_phase(False)

    return body(srcq2d, dst2d, tpflat)


def kernel(x, edge_index, bc_feature, bc_assigment, bset, W1, b1, W2, W4,
           ln_gamma, ln_beta, W_sq):
    n, d = x.shape
    e = edge_index.shape[1]
    nz = bc_feature.shape[0]
    npad = _ceil_to(n, NS * CH)
    epad = _ceil_to(e, NC * NS * CH * 4)
    nxzp = _ceil_to(n + nz, 256)

    src = edge_index[0].astype(i32)
    dst = edge_index[1].astype(i32)
    srcp = jnp.concatenate([src, jnp.full((epad - e,), n, i32)])
    dstp = jnp.concatenate([dst, jnp.full((epad - e,), n, i32)])
    dst2d = dstp.reshape(epad // CH, CH)
    src2de = srcp.reshape(epad // EC, EC)
    dst2de = dstp.reshape(epad // EC, EC)
    srcq2d = jnp.concatenate([src2de * 2, src2de * 2 + 1], axis=0)
    xp = jnp.pad(x.astype(f32), ((0, npad - n), (0, 0)))
    xz = jnp.concatenate([x.astype(f32), bc_feature.astype(f32)], axis=0)
    xzp = jnp.pad(xz, ((0, nxzp - (n + nz)), (0, 0)))
    a0p = jnp.pad(bc_assigment[:n].astype(i32), (0, npad - n))
    a1p = jnp.pad(bc_assigment[n:].astype(i32), (0, npad - n))

    h1 = _matmul(xp, W1.astype(f32))
    h2 = _matmul(xzp, W2.astype(f32))
    deg2 = _sc_deg(dst2d, npad, epad).reshape(NC, npad)
    g0, g1 = _sc_bcgather(h2, a0p, a1p, npad)
    tpack, zp, dinv1 = _tc_b(deg2, h1, g0, g1, W4.astype(f32), npad)
    acc, _ = _sc_edge2(srcq2d, dst2de,
                       tpack.reshape(NC * npad, d // NC), npad, epad)
    acc4 = acc.reshape(2 * NC, npad, d // NC)
    out = _tc_c(acc4, dinv1, h1, zp,
                b1.astype(f32).reshape(1, d),
                ln_gamma.astype(f32).reshape(1, d),
                ln_beta.astype(f32).reshape(1, d),
                W_sq.astype(f32), n)
    return out


# speedup vs baseline: 1.0295x; 1.0295x over previous
import functools

import jax
import jax.numpy as jnp
from jax import lax
from jax.experimental import pallas as pl
from jax.experimental.pallas import tpu as pltpu
from jax.experimental.pallas import tpu_sc as plsc

NC = 2
NS = 16
CH = 128
EC = 64
SB = 16

f32 = jnp.float32
i32 = jnp.int32


def _ceil_to(v, m):
    return (v + m - 1) // m * m


def _sc_mesh():
    return plsc.VectorSubcoreMesh(core_axis_name="c", subcore_axis_name="s")


def _mm_body(a_ref, w_ref, o_ref):
    o_ref[...] = jnp.dot(a_ref[...], w_ref[...], preferred_element_type=f32)


def _matmul(a, w, bm=256):
    m, k = a.shape
    _, n = w.shape
    return pl.pallas_call(
        _mm_body,
        grid=(m // bm,),
        in_specs=[pl.BlockSpec((bm, k), lambda i: (i, 0)),
                  pl.BlockSpec((k, n), lambda i: (0, 0))],
        out_specs=pl.BlockSpec((bm, n), lambda i: (i, 0)),
        out_shape=jax.ShapeDtypeStruct((m, n), f32),
    )(a, w)


def _bf16_bits(x):
    b = lax.bitcast_convert_type(x, jnp.uint32)
    return (b + 0x7FFF + ((b >> 16) & 1)) >> 16


def _pack_pair(lo_f32, hi_f32):
    lo = _bf16_bits(lo_f32)
    hi = _bf16_bits(hi_f32)
    return lax.bitcast_convert_type(lo | (hi << 16), i32)


def _tcb_body(deg_ref, h1_ref, g0_ref, g1_ref, w4_ref, tp_ref, zp_ref,
              di_ref):
    d = h1_ref.shape[1]
    dh = d // NC
    hq = dh // 2
    deg = deg_ref[0] + deg_ref[1] + 1.0
    dinv = lax.rsqrt(deg)[:, None]
    di_ref[...] = dinv
    a = h1_ref[...] * dinv
    z = g0_ref[...] + g1_ref[...]
    zp_ref[...] = z
    b = jnp.dot(z, w4_ref[...], preferred_element_type=f32)
    rows = []
    for cc in range(NC):
        pa = _pack_pair(a[:, cc * dh:cc * dh + hq],
                        a[:, cc * dh + hq:(cc + 1) * dh])
        pb = _pack_pair(b[:, cc * dh:cc * dh + hq],
                        b[:, cc * dh + hq:(cc + 1) * dh])
        rows.append(jnp.concatenate([pa, pb], axis=1))
    tp_ref[...] = jnp.stack(rows, axis=1)


def _tc_b(deg2, h1, g0, g1, w4, npad):
    bm = 256
    d = h1.shape[1]
    dh = d // NC
    return pl.pallas_call(
        _tcb_body,
        grid=(npad // bm,),
        in_specs=[
            pl.BlockSpec((NC, bm), lambda i: (0, i)),
            pl.BlockSpec((bm, d), lambda i: (i, 0)),
            pl.BlockSpec((bm, d), lambda i: (i, 0)),
            pl.BlockSpec((bm, d), lambda i: (i, 0)),
            pl.BlockSpec((d, d), lambda i: (0, 0)),
        ],
        out_specs=[
            pl.BlockSpec((bm, NC, dh), lambda i: (i, 0, 0)),
            pl.BlockSpec((bm, d), lambda i: (i, 0)),
            pl.BlockSpec((bm, 1), lambda i: (i, 0)),
        ],
        out_shape=[
            jax.ShapeDtypeStruct((npad, NC, dh), i32),
            jax.ShapeDtypeStruct((npad, d), f32),
            jax.ShapeDtypeStruct((npad, 1), f32),
        ],
    )(deg2, h1, g0, g1, w4)


def _tcc_body(a0_ref, a1_ref, c0_ref, c1_ref, di_ref, h1_ref, zp_ref,
              b1_ref, gam_ref, bet_ref, wsq_ref, o_ref):
    d = h1_ref.shape[1]
    dinv = di_ref[...]
    xa = jnp.concatenate([a0_ref[0], a1_ref[0]], axis=-1)
    xp = dinv * xa + (dinv * dinv) * h1_ref[...] + b1_ref[...]
    zp = zp_ref[...]
    za = jnp.concatenate([c0_ref[0], c1_ref[0]], axis=-1)
    w = wsq_ref[...]
    gam = gam_ref[...]
    bet = bet_ref[...]

    def ln(t):
        mu = jnp.mean(t, axis=-1, keepdims=True)
        tc = t - mu
        var = jnp.mean(tc * tc, axis=-1, keepdims=True)
        return tc * lax.rsqrt(var + 1e-5) * gam + bet

    acc = jnp.dot(ln(xp), w[0:d], preferred_element_type=f32)
    acc = acc + jnp.dot(ln(zp), w[d:2 * d], preferred_element_type=f32)
    acc = acc + jnp.dot(ln(za), w[2 * d:3 * d], preferred_element_type=f32)
    o_ref[...] = acc


def _tc_c(acc4, dinv1, h1, zp, b1r, gamr, betr, wsq, n):
    br = 200
    d = h1.shape[1]
    dh = d // NC

    def qspec(q):
        return pl.BlockSpec((1, br, dh), lambda i, q=q: (q, i, 0))

    return pl.pallas_call(
        _tcc_body,
        grid=(n // br,),
        in_specs=[
            qspec(0), qspec(1), qspec(2), qspec(3),
            pl.BlockSpec((br, 1), lambda i: (i, 0)),
            pl.BlockSpec((br, d), lambda i: (i, 0)),
            pl.BlockSpec((br, d), lambda i: (i, 0)),
            pl.BlockSpec((1, d), lambda i: (0, 0)),
            pl.BlockSpec((1, d), lambda i: (0, 0)),
            pl.BlockSpec((1, d), lambda i: (0, 0)),
            pl.BlockSpec((3 * d, d), lambda i: (0, 0)),
        ],
        out_specs=pl.BlockSpec((br, d), lambda i: (i, 0)),
        out_shape=jax.ShapeDtypeStruct((n, d), f32),
    )(acc4, acc4, acc4, acc4, dinv1, h1, zp, b1r, gamr, betr, wsq)


def _sc_degbc(dst2d, h2, a0p, a1p, npad, epad):
    per = npad // NS
    nrows = epad // CH
    nchd = nrows // (NC * NS)
    RD = 4
    d = h2.shape[1]
    g = 64
    rpt = npad // (NC * NS)
    jobs_per_stream = rpt // g
    nj = 2 * jobs_per_stream
    R, G = 4, 3

    @functools.partial(
        pl.kernel,
        out_type=[jax.ShapeDtypeStruct((NC * npad,), f32),
                  jax.ShapeDtypeStruct((npad, d), f32),
                  jax.ShapeDtypeStruct((npad, d), f32)],
        mesh=_sc_mesh(),
        scratch_types=[
            pltpu.VMEM((nchd, CH), i32),
            pltpu.VMEM((CH,), f32),
            pltpu.VMEM_SHARED((npad,), f32),
            pltpu.VMEM((rpt,), i32),
            pltpu.VMEM((rpt,), i32),
            pltpu.VMEM((R * g, d), f32),
        ] + [pltpu.SemaphoreType.DMA] * (RD + 2 * R),
    )
    def body(dst_hbm, h2_hbm, a0_hbm, a1_hbm, deg_hbm, g0_hbm, g1_hbm,
             didx, ones_v, acc_sh, i0, i1, ring,
             d0, d1, d2, d3, ga0, ga1, ga2, ga3, wa0, wa1, wa2, wa3):
        c = lax.axis_index("c")
        s = lax.axis_index("s")
        dsems = (d0, d1, d2, d3)
        gsems = (ga0, ga1, ga2, ga3)
        wsems = (wa0, wa1, wa2, wa3)
        w = c * NS + s
        wg = s * NC + c

        pltpu.sync_copy(dst_hbm.at[pl.ds(w * nchd, nchd), :], didx)
        for q in range(CH // 16):
            ones_v[pl.ds(q * 16, 16)] = jnp.zeros((16,), f32)
        for kk in range(per // CH):
            pltpu.sync_copy(ones_v, acc_sh.at[pl.ds(s * per + kk * CH, CH)])
        for q in range(CH // 16):
            ones_v[pl.ds(q * 16, 16)] = jnp.ones((16,), f32)
        plsc.subcore_barrier()

        def dfire(r, j):
            pltpu.async_copy(ones_v, acc_sh.at[didx.at[j]], dsems[r], add=True)

        def dwait(r):
            pltpu.make_async_copy(ones_v, acc_sh.at[didx.at[0]],
                                  dsems[r]).wait()

        def dstep(j4, carry):
            for q in range(RD):
                j = j4 * RD + q

                @pl.when(j4 > 0)
                def _():
                    dwait(q)

                dfire(q, j)
            return carry

        lax.fori_loop(0, nchd // RD, dstep, 0)
        for r in range(RD):
            dwait(r)
        plsc.subcore_barrier()
        pltpu.sync_copy(acc_sh.at[pl.ds(s * per, per)],
                        deg_hbm.at[pl.ds(c * npad + s * per, per)])

        pltpu.sync_copy(a0_hbm.at[pl.ds(wg * rpt, rpt)], i0)
        pltpu.sync_copy(a1_hbm.at[pl.ds(wg * rpt, rpt)], i1)

        def slot(r):
            return ring.at[pl.ds(r * g, g), :]

        def job_refs(j):
            if j < jobs_per_stream:
                return i0.at[pl.ds(j * g, g)], g0_hbm, j
            return i1.at[pl.ds((j - jobs_per_stream) * g, g)], g1_hbm, \
                j - jobs_per_stream

        def fire_gather(r, j):
            idx, _, _ = job_refs(j)
            pltpu.async_copy(h2_hbm.at[idx], slot(r), gsems[r])

        def wait_gather(r, j):
            idx, _, _ = job_refs(j)
            pltpu.make_async_copy(h2_hbm.at[idx], slot(r), gsems[r]).wait()

        def out_rows(j):
            _, out, jj = job_refs(j)
            return out.at[pl.ds(wg * rpt + jj * g, g), :]

        def fire_write(r, j):
            pltpu.async_copy(slot(r), out_rows(j), wsems[r])

        def wait_write(r, j):
            pltpu.make_async_copy(slot(r), out_rows(j), wsems[r]).wait()

        for j in range(G):
            fire_gather(j % R, j)
        for j in range(nj):
            r = j % R
            wait_gather(r, j)
            fire_write(r, j)
            jn = j + G
            if jn < nj:
                rn = jn % R
                if jn >= R:
                    wait_write(rn, jn - R)
                fire_gather(rn, jn)
        for j in range(nj - R, nj):
            wait_write(j % R, j)

    return body(dst2d, h2, a0p, a1p)


def _sc_edge2(srcq2d, dst2d, tpflat, npad, epad):
    dh = 128
    hq = dh // 2
    per = npad // NS
    nrows = epad // EC
    nch = nrows // NS
    nsup = nch // SB
    R, G = 4, 3

    @functools.partial(
        pl.kernel,
        out_type=[jax.ShapeDtypeStruct((2 * NC * npad, dh), f32),
                  jax.ShapeDtypeStruct((NC * epad, dh), i32)],
        mesh=_sc_mesh(),
        scratch_types=[
            pltpu.VMEM((2 * SB, EC), i32),
            pltpu.VMEM((2 * SB, EC), i32),
            pltpu.VMEM((R * EC, dh), i32),
            pltpu.VMEM((EC, dh), f32),
            pltpu.VMEM_SHARED((npad, dh), f32),
        ] + [pltpu.SemaphoreType.DMA] * 13,
    )
    def body(srcq_hbm, dst_hbm, tp_hbm, out_hbm, sp_hbm,
             gidx, didx, ring, conv, acc_sh,
             g0, g1, g2, g3, p0, p1, p2, p3, sc0, ig0, ig1, id0, id1):
        c = lax.axis_index("c")
        s = lax.axis_index("s")
        gsems = (g0, g1, g2, g3)
        psems = (p0, p1, p2, p3)
        igsems = (ig0, ig1)
        idsems = (id0, id1)

        def slot(r):
            return ring.at[pl.ds(r * EC, EC), :]

        def spill_rows(j):
            return sp_hbm.at[pl.ds(c * epad + (s * nch + j) * EC, EC), :]

        def fire_spill(r, j):
            pltpu.async_copy(slot(r), spill_rows(j), psems[r])

        def wait_spill(r, j):
            pltpu.make_async_copy(slot(r), spill_rows(j), psems[r]).wait()

        def fire_scatter(row_sel):
            pltpu.async_copy(conv, acc_sh.at[didx.at[row_sel]], sc0, add=True)

        def wait_scatter(row_sel):
            pltpu.make_async_copy(conv, acc_sh.at[didx.at[row_sel]],
                                  sc0).wait()

        def unpack(r, lane0):
            def row(i, carry):
                for q in range(hq // 16):
                    v = ring[r * EC + i, pl.ds(lane0 + q * 16, 16)]
                    lo = lax.bitcast_convert_type(v << 16, f32)
                    hi = lax.bitcast_convert_type(v & jnp.int32(-65536), f32)
                    conv[i, pl.ds(q * 16, 16)] = lo
                    conv[i, pl.ds(hq + q * 16, 16)] = hi
                return carry

            lax.fori_loop(0, EC, row, 0)

        def zero_acc():
            def zrow(i, carry):
                for qq in range(dh // 16):
                    conv[i, pl.ds(qq * 16, 16)] = jnp.zeros((16,), f32)
                return carry

            lax.fori_loop(0, EC, zrow, 0)
            for kk in range(per // EC):
                pltpu.sync_copy(conv,
                                acc_sh.at[pl.ds(s * per + kk * EC, EC), :])
            plsc.subcore_barrier()

        def stage_idx(b, u, sync, phase1):
            ds_ = dst_hbm.at[pl.ds(s * nch + u * SB, SB), :]
            dv = didx.at[pl.ds(b * SB, SB), :]
            if sync:
                pltpu.sync_copy(ds_, dv)
            else:
                pltpu.async_copy(ds_, dv, idsems[b])
            if phase1:
                gs = srcq_hbm.at[pl.ds(c * nrows + s * nch + u * SB, SB), :]
                gv = gidx.at[pl.ds(b * SB, SB), :]
                if sync:
                    pltpu.sync_copy(gs, gv)
                else:
                    pltpu.async_copy(gs, gv, igsems[b])

        def wait_idx(b, phase1):
            ds_ = dst_hbm.at[pl.ds(s * nch, SB), :]
            dv = didx.at[pl.ds(b * SB, SB), :]
            pltpu.make_async_copy(ds_, dv, idsems[b]).wait()
            if phase1:
                gs = srcq_hbm.at[pl.ds(c * nrows, SB), :]
                gv = gidx.at[pl.ds(b * SB, SB), :]
                pltpu.make_async_copy(gs, gv, igsems[b]).wait()

        def fire_fetch(r, j, row_sel, phase1):
            if phase1:
                pltpu.async_copy(tp_hbm.at[gidx.at[row_sel]], slot(r),
                                 gsems[r])
            else:
                pltpu.async_copy(spill_rows(j), slot(r), gsems[r])

        def wait_fetch(r, j, row_sel, phase1):
            if phase1:
                pltpu.make_async_copy(tp_hbm.at[gidx.at[row_sel]], slot(r),
                                      gsems[r]).wait()
            else:
                pltpu.make_async_copy(spill_rows(j), slot(r), gsems[r]).wait()

        def run_phase(phase1):
            k = (0 if phase1 else NC) + c
            lane0 = 0 if phase1 else hq
            zero_acc()
            stage_idx(0, 0, True, phase1)
            for j in range(G):
                fire_fetch(j % R, j, j, phase1)

            def pair_step(u2, carry):
                for half in range(2):
                    u = u2 * 2 + half
                    b = half
                    bn = 1 - half
                    for jj in range(SB):
                        j = u * SB + jj
                        r = jj % R
                        row = b * SB + jj
                        if jj == 2:
                            @pl.when(u + 1 < nsup)
                            def _():
                                stage_idx(bn, u + 1, False, phase1)
                        wait_fetch(r, j, row, phase1)
                        fire_spill_maybe = phase1
                        if fire_spill_maybe:
                            fire_spill(r, j)

                        @pl.when(j > 0)
                        def _():
                            wait_scatter(row)

                        unpack(r, lane0)
                        fire_scatter(row)
                        jn = j + G
                        rn = (jj + G) % R
                        if jj < SB - G:
                            rown = b * SB + (jj + G)
                            crosses = False
                        else:
                            rown = bn * SB + (jj + G - SB)
                            crosses = True
                        if crosses:
                            @pl.when(jn < nch)
                            def _():
                                if phase1:
                                    wait_spill(rn, jn)
                                if jj == SB - G:
                                    wait_idx(bn, phase1)
                                fire_fetch(rn, jn, rown, phase1)
                        else:
                            @pl.when(jn >= R)
                            def _():
                                if phase1:
                                    wait_spill(rn, jn)

                            fire_fetch(rn, jn, rown, phase1)
                return carry

            lax.fori_loop(0, nsup // 2, pair_step, 0)
            wait_scatter(0)
            if phase1:
                for r in range(R):
                    wait_spill(r, 0)
            plsc.subcore_barrier()
            pltpu.sync_copy(acc_sh.at[pl.ds(s * per, per), :],
                            out_hbm.at[pl.ds(k * npad + s * per, per), :])

        run_phase(True)
        run_phase(False)

    return body(srcq2d, dst2d, tpflat)


def kernel(x, edge_index, bc_feature, bc_assigment, bset, W1, b1, W2, W4,
           ln_gamma, ln_beta, W_sq):
    n, d = x.shape
    e = edge_index.shape[1]
    nz = bc_feature.shape[0]
    npad = _ceil_to(n, NS * CH)
    epad = _ceil_to(e, NC * NS * CH * 4)
    nxzp = _ceil_to(n + nz, 256)

    src = edge_index[0].astype(i32)
    dst = edge_index[1].astype(i32)
    srcp = jnp.concatenate([src, jnp.full((epad - e,), n, i32)])
    dstp = jnp.concatenate([dst, jnp.full((epad - e,), n, i32)])
    dst2d = dstp.reshape(epad // CH, CH)
    src2de = srcp.reshape(epad // EC, EC)
    dst2de = dstp.reshape(epad // EC, EC)
    srcq2d = jnp.concatenate([src2de * 2, src2de * 2 + 1], axis=0)
    xp = jnp.pad(x.astype(f32), ((0, npad - n), (0, 0)))
    xz = jnp.concatenate([x.astype(f32), bc_feature.astype(f32)], axis=0)
    xzp = jnp.pad(xz, ((0, nxzp - (n + nz)), (0, 0)))
    a0p = jnp.pad(bc_assigment[:n].astype(i32), (0, npad - n))
    a1p = jnp.pad(bc_assigment[n:].astype(i32), (0, npad - n))

    h1 = _matmul(xp, W1.astype(f32))
    h2 = _matmul(xzp, W2.astype(f32))
    deg2r, g0, g1 = _sc_degbc(dst2d, h2, a0p, a1p, npad, epad)
    deg2 = deg2r.reshape(NC, npad)
    tpack, zp, dinv1 = _tc_b(deg2, h1, g0, g1, W4.astype(f32), npad)
    acc, _ = _sc_edge2(srcq2d, dst2de,
                       tpack.reshape(NC * npad, d // NC), npad, epad)
    acc4 = acc.reshape(2 * NC, npad, d // NC)
    out = _tc_c(acc4, dinv1, h1, zp,
                b1.astype(f32).reshape(1, d),
                ln_gamma.astype(f32).reshape(1, d),
                ln_beta.astype(f32).reshape(1, d),
                W_sq.astype(f32), n)
    return out


# speedup vs baseline: 1.0484x; 1.0184x over previous
import functools

import jax
import jax.numpy as jnp
from jax import lax
from jax.experimental import pallas as pl
from jax.experimental.pallas import tpu as pltpu
from jax.experimental.pallas import tpu_sc as plsc

NC = 2
NS = 16
CH = 128
EC = 64
SB = 16

f32 = jnp.float32
i32 = jnp.int32


def _ceil_to(v, m):
    return (v + m - 1) // m * m


def _sc_mesh():
    return plsc.VectorSubcoreMesh(core_axis_name="c", subcore_axis_name="s")


def _mm_body(a_ref, w_ref, o_ref):
    o_ref[...] = jnp.dot(a_ref[...], w_ref[...], preferred_element_type=f32)


def _matmul(a, w, bm=512):
    m, k = a.shape
    _, n = w.shape
    return pl.pallas_call(
        _mm_body,
        grid=(m // bm,),
        in_specs=[pl.BlockSpec((bm, k), lambda i: (i, 0)),
                  pl.BlockSpec((k, n), lambda i: (0, 0))],
        out_specs=pl.BlockSpec((bm, n), lambda i: (i, 0)),
        out_shape=jax.ShapeDtypeStruct((m, n), f32),
    )(a, w)


def _bf16_bits(x):
    b = lax.bitcast_convert_type(x, jnp.uint32)
    return (b + 0x7FFF + ((b >> 16) & 1)) >> 16


def _pack_pair(lo_f32, hi_f32):
    lo = _bf16_bits(lo_f32)
    hi = _bf16_bits(hi_f32)
    return lax.bitcast_convert_type(lo | (hi << 16), i32)


def _tcb_body(deg_ref, h1_ref, g0_ref, g1_ref, w4_ref, tp_ref, zp_ref,
              di_ref):
    d = h1_ref.shape[1]
    dh = d // NC
    hq = dh // 2
    deg = deg_ref[0] + deg_ref[1] + 1.0
    dinv = lax.rsqrt(deg)[:, None]
    di_ref[...] = dinv
    a = h1_ref[...] * dinv
    z = g0_ref[...] + g1_ref[...]
    zp_ref[...] = z
    b = jnp.dot(z, w4_ref[...], preferred_element_type=f32)
    rows = []
    for cc in range(NC):
        pa = _pack_pair(a[:, cc * dh:cc * dh + hq],
                        a[:, cc * dh + hq:(cc + 1) * dh])
        pb = _pack_pair(b[:, cc * dh:cc * dh + hq],
                        b[:, cc * dh + hq:(cc + 1) * dh])
        rows.append(jnp.concatenate([pa, pb], axis=1))
    tp_ref[...] = jnp.stack(rows, axis=1)


def _tc_b(deg2, h1, g0, g1, w4, npad):
    bm = 256
    d = h1.shape[1]
    dh = d // NC
    return pl.pallas_call(
        _tcb_body,
        grid=(npad // bm,),
        in_specs=[
            pl.BlockSpec((NC, bm), lambda i: (0, i)),
            pl.BlockSpec((bm, d), lambda i: (i, 0)),
            pl.BlockSpec((bm, d), lambda i: (i, 0)),
            pl.BlockSpec((bm, d), lambda i: (i, 0)),
            pl.BlockSpec((d, d), lambda i: (0, 0)),
        ],
        out_specs=[
            pl.BlockSpec((bm, NC, dh), lambda i: (i, 0, 0)),
            pl.BlockSpec((bm, d), lambda i: (i, 0)),
            pl.BlockSpec((bm, 1), lambda i: (i, 0)),
        ],
        out_shape=[
            jax.ShapeDtypeStruct((npad, NC, dh), i32),
            jax.ShapeDtypeStruct((npad, d), f32),
            jax.ShapeDtypeStruct((npad, 1), f32),
        ],
    )(deg2, h1, g0, g1, w4)


def _tcc_body(a0_ref, a1_ref, c0_ref, c1_ref, di_ref, h1_ref, zp_ref,
              b1_ref, gam_ref, bet_ref, wsq_ref, o_ref):
    d = h1_ref.shape[1]
    dinv = di_ref[...]
    xa = jnp.concatenate([a0_ref[0], a1_ref[0]], axis=-1)
    xp = dinv * xa + (dinv * dinv) * h1_ref[...] + b1_ref[...]
    zp = zp_ref[...]
    za = jnp.concatenate([c0_ref[0], c1_ref[0]], axis=-1)
    w = wsq_ref[...]
    gam = gam_ref[...]
    bet = bet_ref[...]

    def ln(t):
        mu = jnp.mean(t, axis=-1, keepdims=True)
        tc = t - mu
        var = jnp.mean(tc * tc, axis=-1, keepdims=True)
        return tc * lax.rsqrt(var + 1e-5) * gam + bet

    acc = jnp.dot(ln(xp), w[0:d], preferred_element_type=f32)
    acc = acc + jnp.dot(ln(zp), w[d:2 * d], preferred_element_type=f32)
    acc = acc + jnp.dot(ln(za), w[2 * d:3 * d], preferred_element_type=f32)
    o_ref[...] = acc


def _tc_c(acc4, dinv1, h1, zp, b1r, gamr, betr, wsq, n):
    br = 200
    d = h1.shape[1]
    dh = d // NC

    def qspec(q):
        return pl.BlockSpec((1, br, dh), lambda i, q=q: (q, i, 0))

    return pl.pallas_call(
        _tcc_body,
        grid=(n // br,),
        in_specs=[
            qspec(0), qspec(1), qspec(2), qspec(3),
            pl.BlockSpec((br, 1), lambda i: (i, 0)),
            pl.BlockSpec((br, d), lambda i: (i, 0)),
            pl.BlockSpec((br, d), lambda i: (i, 0)),
            pl.BlockSpec((1, d), lambda i: (0, 0)),
            pl.BlockSpec((1, d), lambda i: (0, 0)),
            pl.BlockSpec((1, d), lambda i: (0, 0)),
            pl.BlockSpec((3 * d, d), lambda i: (0, 0)),
        ],
        out_specs=pl.BlockSpec((br, d), lambda i: (i, 0)),
        out_shape=jax.ShapeDtypeStruct((n, d), f32),
    )(acc4, acc4, acc4, acc4, dinv1, h1, zp, b1r, gamr, betr, wsq)


def _sc_degbc(dst2d, h2, a0p, a1p, npad, epad):
    per = npad // NS
    nrows = epad // CH
    nchd = nrows // (NC * NS)
    RD = 4
    d = h2.shape[1]
    g = 64
    rpt = npad // (NC * NS)
    jobs_per_stream = rpt // g
    nj = 2 * jobs_per_stream
    R, G = 4, 3

    @functools.partial(
        pl.kernel,
        out_type=[jax.ShapeDtypeStruct((NC * npad,), f32),
                  jax.ShapeDtypeStruct((npad, d), f32),
                  jax.ShapeDtypeStruct((npad, d), f32)],
        mesh=_sc_mesh(),
        scratch_types=[
            pltpu.VMEM((nchd, CH), i32),
            pltpu.VMEM((CH,), f32),
            pltpu.VMEM_SHARED((npad,), f32),
            pltpu.VMEM((rpt,), i32),
            pltpu.VMEM((rpt,), i32),
            pltpu.VMEM((R * g, d), f32),
        ] + [pltpu.SemaphoreType.DMA] * (RD + 2 * R),
    )
    def body(dst_hbm, h2_hbm, a0_hbm, a1_hbm, deg_hbm, g0_hbm, g1_hbm,
             didx, ones_v, acc_sh, i0, i1, ring,
             d0, d1, d2, d3, ga0, ga1, ga2, ga3, wa0, wa1, wa2, wa3):
        c = lax.axis_index("c")
        s = lax.axis_index("s")
        dsems = (d0, d1, d2, d3)
        gsems = (ga0, ga1, ga2, ga3)
        wsems = (wa0, wa1, wa2, wa3)
        w = c * NS + s
        wg = s * NC + c

        pltpu.sync_copy(dst_hbm.at[pl.ds(w * nchd, nchd), :], didx)
        for q in range(CH // 16):
            ones_v[pl.ds(q * 16, 16)] = jnp.zeros((16,), f32)
        for kk in range(per // CH):
            pltpu.sync_copy(ones_v, acc_sh.at[pl.ds(s * per + kk * CH, CH)])
        for q in range(CH // 16):
            ones_v[pl.ds(q * 16, 16)] = jnp.ones((16,), f32)
        plsc.subcore_barrier()

        def dfire(r, j):
            pltpu.async_copy(ones_v, acc_sh.at[didx.at[j]], dsems[r], add=True)

        def dwait(r):
            pltpu.make_async_copy(ones_v, acc_sh.at[didx.at[0]],
                                  dsems[r]).wait()

        def dstep(j4, carry):
            for q in range(RD):
                j = j4 * RD + q

                @pl.when(j4 > 0)
                def _():
                    dwait(q)

                dfire(q, j)
            return carry

        lax.fori_loop(0, nchd // RD, dstep, 0)
        for r in range(RD):
            dwait(r)
        plsc.subcore_barrier()
        pltpu.sync_copy(acc_sh.at[pl.ds(s * per, per)],
                        deg_hbm.at[pl.ds(c * npad + s * per, per)])

        pltpu.sync_copy(a0_hbm.at[pl.ds(wg * rpt, rpt)], i0)
        pltpu.sync_copy(a1_hbm.at[pl.ds(wg * rpt, rpt)], i1)

        def slot(r):
            return ring.at[pl.ds(r * g, g), :]

        def job_refs(j):
            if j < jobs_per_stream:
                return i0.at[pl.ds(j * g, g)], g0_hbm, j
            return i1.at[pl.ds((j - jobs_per_stream) * g, g)], g1_hbm, \
                j - jobs_per_stream

        def fire_gather(r, j):
            idx, _, _ = job_refs(j)
            pltpu.async_copy(h2_hbm.at[idx], slot(r), gsems[r])

        def wait_gather(r, j):
            idx, _, _ = job_refs(j)
            pltpu.make_async_copy(h2_hbm.at[idx], slot(r), gsems[r]).wait()

        def out_rows(j):
            _, out, jj = job_refs(j)
            return out.at[pl.ds(wg * rpt + jj * g, g), :]

        def fire_write(r, j):
            pltpu.async_copy(slot(r), out_rows(j), wsems[r])

        def wait_write(r, j):
            pltpu.make_async_copy(slot(r), out_rows(j), wsems[r]).wait()

        for j in range(G):
            fire_gather(j % R, j)
        for j in range(nj):
            r = j % R
            wait_gather(r, j)
            fire_write(r, j)
            jn = j + G
            if jn < nj:
                rn = jn % R
                if jn >= R:
                    wait_write(rn, jn - R)
                fire_gather(rn, jn)
        for j in range(nj - R, nj):
            wait_write(j % R, j)

    return body(dst2d, h2, a0p, a1p)


def _sc_edge2(srcq2d, dst2d, tpflat, npad, epad):
    dh = 128
    hq = dh // 2
    per = npad // NS
    nrows = epad // EC
    nch = nrows // NS
    nsup = nch // SB
    R, G = 4, 3

    @functools.partial(
        pl.kernel,
        out_type=[jax.ShapeDtypeStruct((2 * NC * npad, dh), f32),
                  jax.ShapeDtypeStruct((NC * epad, dh), i32)],
        mesh=_sc_mesh(),
        scratch_types=[
            pltpu.VMEM((2 * SB, EC), i32),
            pltpu.VMEM((2 * SB, EC), i32),
            pltpu.VMEM((R * EC, dh), i32),
            pltpu.VMEM((EC, dh), f32),
            pltpu.VMEM_SHARED((npad, dh), f32),
        ] + [pltpu.SemaphoreType.DMA] * 13,
    )
    def body(srcq_hbm, dst_hbm, tp_hbm, out_hbm, sp_hbm,
             gidx, didx, ring, conv, acc_sh,
             g0, g1, g2, g3, p0, p1, p2, p3, sc0, ig0, ig1, id0, id1):
        c = lax.axis_index("c")
        s = lax.axis_index("s")
        gsems = (g0, g1, g2, g3)
        psems = (p0, p1, p2, p3)
        igsems = (ig0, ig1)
        idsems = (id0, id1)

        def slot(r):
            return ring.at[pl.ds(r * EC, EC), :]

        def spill_rows(j):
            return sp_hbm.at[pl.ds(c * epad + (s * nch + j) * EC, EC), :]

        def fire_spill(r, j):
            pltpu.async_copy(slot(r), spill_rows(j), psems[r])

        def wait_spill(r, j):
            pltpu.make_async_copy(slot(r), spill_rows(j), psems[r]).wait()

        def fire_scatter(row_sel):
            pltpu.async_copy(conv, acc_sh.at[didx.at[row_sel]], sc0, add=True)

        def wait_scatter(row_sel):
            pltpu.make_async_copy(conv, acc_sh.at[didx.at[row_sel]],
                                  sc0).wait()

        def unpack(r, lane0):
            def row(i, carry):
                for q in range(hq // 16):
                    v = ring[r * EC + i, pl.ds(lane0 + q * 16, 16)]
                    lo = lax.bitcast_convert_type(v << 16, f32)
                    hi = lax.bitcast_convert_type(v & jnp.int32(-65536), f32)
                    conv[i, pl.ds(q * 16, 16)] = lo
                    conv[i, pl.ds(hq + q * 16, 16)] = hi
                return carry

            lax.fori_loop(0, EC, row, 0)

        def zero_acc():
            def zrow(i, carry):
                for qq in range(dh // 16):
                    conv[i, pl.ds(qq * 16, 16)] = jnp.zeros((16,), f32)
                return carry

            lax.fori_loop(0, EC, zrow, 0)
            for kk in range(per // EC):
                pltpu.sync_copy(conv,
                                acc_sh.at[pl.ds(s * per + kk * EC, EC), :])
            plsc.subcore_barrier()

        def stage_idx(b, u, sync, phase1):
            ds_ = dst_hbm.at[pl.ds(s * nch + u * SB, SB), :]
            dv = didx.at[pl.ds(b * SB, SB), :]
            if sync:
                pltpu.sync_copy(ds_, dv)
            else:
                pltpu.async_copy(ds_, dv, idsems[b])
            if phase1:
                gs = srcq_hbm.at[pl.ds(c * nrows + s * nch + u * SB, SB), :]
                gv = gidx.at[pl.ds(b * SB, SB), :]
                if sync:
                    pltpu.sync_copy(gs, gv)
                else:
                    pltpu.async_copy(gs, gv, igsems[b])

        def wait_idx(b, phase1):
            ds_ = dst_hbm.at[pl.ds(s * nch, SB), :]
            dv = didx.at[pl.ds(b * SB, SB), :]
            pltpu.make_async_copy(ds_, dv, idsems[b]).wait()
            if phase1:
                gs = srcq_hbm.at[pl.ds(c * nrows, SB), :]
                gv = gidx.at[pl.ds(b * SB, SB), :]
                pltpu.make_async_copy(gs, gv, igsems[b]).wait()

        def fire_fetch(r, j, row_sel, phase1):
            if phase1:
                pltpu.async_copy(tp_hbm.at[gidx.at[row_sel]], slot(r),
                                 gsems[r])
            else:
                pltpu.async_copy(spill_rows(j), slot(r), gsems[r])

        def wait_fetch(r, j, row_sel, phase1):
            if phase1:
                pltpu.make_async_copy(tp_hbm.at[gidx.at[row_sel]], slot(r),
                                      gsems[r]).wait()
            else:
                pltpu.make_async_copy(spill_rows(j), slot(r), gsems[r]).wait()

        def run_phase(phase1):
            k = (0 if phase1 else NC) + c
            lane0 = 0 if phase1 else hq
            zero_acc()
            stage_idx(0, 0, True, phase1)
            for j in range(G):
                fire_fetch(j % R, j, j, phase1)

            def pair_step(u2, carry):
                for half in range(2):
                    u = u2 * 2 + half
                    b = half
                    bn = 1 - half
                    for jj in range(SB):
                        j = u * SB + jj
                        r = jj % R
                        row = b * SB + jj
                        if jj == 2:
                            @pl.when(u + 1 < nsup)
                            def _():
                                stage_idx(bn, u + 1, False, phase1)
                        wait_fetch(r, j, row, phase1)
                        fire_spill_maybe = phase1
                        if fire_spill_maybe:
                            fire_spill(r, j)

                        @pl.when(j > 0)
                        def _():
                            wait_scatter(row)

                        unpack(r, lane0)
                        fire_scatter(row)
                        jn = j + G
                        rn = (jj + G) % R
                        if jj < SB - G:
                            rown = b * SB + (jj + G)
                            crosses = False
                        else:
                            rown = bn * SB + (jj + G - SB)
                            crosses = True
                        if crosses:
                            @pl.when(jn < nch)
                            def _():
                                if phase1:
                                    wait_spill(rn, jn)
                                if jj == SB - G:
                                    wait_idx(bn, phase1)
                                fire_fetch(rn, jn, rown, phase1)
                        else:
                            @pl.when(jn >= R)
                            def _():
                                if phase1:
                                    wait_spill(rn, jn)

                            fire_fetch(rn, jn, rown, phase1)
                return carry

            lax.fori_loop(0, nsup // 2, pair_step, 0)
            wait_scatter(0)
            if phase1:
                for r in range(R):
                    wait_spill(r, 0)
            plsc.subcore_barrier()
            pltpu.sync_copy(acc_sh.at[pl.ds(s * per, per), :],
                            out_hbm.at[pl.ds(k * npad + s * per, per), :])

        run_phase(True)
        run_phase(False)

    return body(srcq2d, dst2d, tpflat)


def kernel(x, edge_index, bc_feature, bc_assigment, bset, W1, b1, W2, W4,
           ln_gamma, ln_beta, W_sq):
    n, d = x.shape
    e = edge_index.shape[1]
    nz = bc_feature.shape[0]
    npad = _ceil_to(n, NS * CH)
    epad = _ceil_to(e, NC * NS * CH * 4)
    nxzp = _ceil_to(n + nz, 256)

    src = edge_index[0].astype(i32)
    dst = edge_index[1].astype(i32)
    srcp = jnp.concatenate([src, jnp.full((epad - e,), n, i32)])
    dstp = jnp.concatenate([dst, jnp.full((epad - e,), n, i32)])
    dst2d = dstp.reshape(epad // CH, CH)
    src2de = srcp.reshape(epad // EC, EC)
    dst2de = dstp.reshape(epad // EC, EC)
    srcq2d = jnp.concatenate([src2de * 2, src2de * 2 + 1], axis=0)
    xp = jnp.pad(x.astype(f32), ((0, npad - n), (0, 0)))
    xz = jnp.concatenate([x.astype(f32), bc_feature.astype(f32)], axis=0)
    xzp = jnp.pad(xz, ((0, nxzp - (n + nz)), (0, 0)))
    a0p = jnp.pad(bc_assigment[:n].astype(i32), (0, npad - n))
    a1p = jnp.pad(bc_assigment[n:].astype(i32), (0, npad - n))

    h1 = _matmul(xp, W1.astype(f32))
    h2 = _matmul(xzp, W2.astype(f32))
    deg2r, g0, g1 = _sc_degbc(dst2d, h2, a0p, a1p, npad, epad)
    deg2 = deg2r.reshape(NC, npad)
    tpack, zp, dinv1 = _tc_b(deg2, h1, g0, g1, W4.astype(f32), npad)
    acc, _ = _sc_edge2(srcq2d, dst2de,
                       tpack.reshape(NC * npad, d // NC), npad, epad)
    acc4 = acc.reshape(2 * NC, npad, d // NC)
    out = _tc_c(acc4, dinv1, h1, zp,
                b1.astype(f32).reshape(1, d),
                ln_gamma.astype(f32).reshape(1, d),
                ln_beta.astype(f32).reshape(1, d),
                W_sq.astype(f32), n)
    return out
